# Initial kernel scaffold; baseline (speedup 1.0000x reference)
import jax
import jax.numpy as jnp
from jax.experimental import pallas as pl


def kernel(x, pos, edge_index, W1, W2, W3):
    raise NotImplementedError("write your pallas kernel here")


# speedup vs baseline: 1.6931x; 1.6931x over previous
import functools

import jax
import jax.numpy as jnp
import numpy as np
from jax import lax
from jax.experimental import pallas as pl
from jax.experimental.pallas import tpu as pltpu

N_NODES = 10000
E_EDGES = 160000
MUL_IN = 32
MUL_OUT = 8
D_IN = 128
D_OUT = 32
NUM_RBF = 24
R_CUT = 10.0
HIDDEN = 64
BE = 2000

_SQRT3 = np.sqrt(3.0).astype(np.float32)
_ALPHA = np.float32(1.0 / np.sqrt(2 * MUL_IN))


def _edge_body(posr_ref, posc_ref, xg_ref, w1_ref, w2_ref, w3_ref, msg_ref):
    f32 = jnp.float32
    posr = posr_ref[...]
    posc = posc_ref[...]
    vec = posr - posc
    r2 = jnp.sum(vec * vec, axis=1, keepdims=True)
    r = jnp.sqrt(r2)
    n = vec * (1.0 / (r + 1e-9))
    y1_0 = _SQRT3 * n[:, 1:2]
    y1_1 = _SQRT3 * n[:, 2:3]
    y1_2 = _SQRT3 * n[:, 0:1]

    spacing = R_CUT / (NUM_RBF - 1)
    offs = lax.broadcasted_iota(jnp.int32, (1, NUM_RBF), 1).astype(f32) * spacing
    t = (r - offs) * (1.0 / spacing)
    phi = jnp.exp(-0.5 * t * t)

    h = phi @ w1_ref[...]
    h = h * jax.nn.sigmoid(h)
    h = h @ w2_ref[...]
    h = h * jax.nn.sigmoid(h)
    w = h @ w3_ref[...]

    xs = xg_ref[:, 0:32]
    xv0 = xg_ref[:, 32:64]
    xv1 = xg_ref[:, 64:96]
    xv2 = xg_ref[:, 96:128]
    s1 = (xv0 * y1_0 + xv1 * y1_1 + xv2 * y1_2) * (1.0 / _SQRT3)

    iu = lax.broadcasted_iota(jnp.int32, (MUL_IN, MUL_IN * MUL_OUT), 0)
    ij = lax.broadcasted_iota(jnp.int32, (MUL_IN, MUL_IN * MUL_OUT), 1)
    Q = (ij // MUL_OUT == iu).astype(f32)
    pj = lax.broadcasted_iota(jnp.int32, (MUL_IN * MUL_OUT, MUL_OUT), 0)
    pw = lax.broadcasted_iota(jnp.int32, (MUL_IN * MUL_OUT, MUL_OUT), 1)
    P8 = (pj % MUL_OUT == pw).astype(f32)

    s0r = xs @ Q
    s1r = s1 @ Q
    w0 = w[:, 0:256]
    w1p = w[:, 256:512]
    w2p = w[:, 512:768]
    w3p = w[:, 768:1024]

    out_s = (w0 * s0r + w3p * s1r) @ P8
    va = (w1p * s0r) @ P8
    vb0 = (w2p * (xv0 @ Q)) @ P8
    vb1 = (w2p * (xv1 @ Q)) @ P8
    vb2 = (w2p * (xv2 @ Q)) @ P8
    vk = jnp.concatenate(
        [va * y1_0 + vb0, va * y1_1 + vb1, va * y1_2 + vb2], axis=1
    )
    sk = lax.broadcasted_iota(jnp.int32, (24, 24), 0)
    sc = lax.broadcasted_iota(jnp.int32, (24, 24), 1)
    S = ((sc % 3) * 8 + sc // 3 == sk).astype(f32)
    msg_v = vk @ S
    msg_ref[...] = jnp.concatenate([out_s, msg_v], axis=1)


@jax.jit
def _edge_tc(posr, posc, xg, w1c, w2c, w3c):
    grid = (E_EDGES // BE,)
    return pl.pallas_call(
        _edge_body,
        grid=grid,
        in_specs=[
            pl.BlockSpec((BE, 8), lambda i: (i, 0)),
            pl.BlockSpec((BE, 8), lambda i: (i, 0)),
            pl.BlockSpec((BE, D_IN), lambda i: (i, 0)),
            pl.BlockSpec((NUM_RBF, HIDDEN), lambda i: (0, 0)),
            pl.BlockSpec((HIDDEN, HIDDEN), lambda i: (0, 0)),
            pl.BlockSpec((HIDDEN, 4 * MUL_IN * MUL_OUT), lambda i: (0, 0)),
        ],
        out_specs=pl.BlockSpec((BE, D_OUT), lambda i: (i, 0)),
        out_shape=jax.ShapeDtypeStruct((E_EDGES, D_OUT), jnp.float32),
        compiler_params=pltpu.CompilerParams(
            dimension_semantics=("arbitrary",),
        ),
    )(posr, posc, xg, w1c, w2c, w3c)


def _act_body(o_ref, out_ref):
    o = o_ref[...]
    s = o[:, :MUL_OUT]
    out_ref[...] = jnp.concatenate(
        [s * jax.nn.sigmoid(s), o[:, MUL_OUT:]], axis=1
    )


@jax.jit
def _act_tc(o):
    return pl.pallas_call(
        _act_body,
        out_shape=jax.ShapeDtypeStruct((N_NODES, D_OUT), jnp.float32),
    )(o)


def kernel(x, pos, edge_index, W1, W2, W3):
    row = edge_index[0]
    col = edge_index[1]
    xs = x[:, :MUL_IN]
    xv = x[:, MUL_IN:].reshape(N_NODES, MUL_IN, 3)
    xT = jnp.concatenate([xs, xv[:, :, 0], xv[:, :, 1], xv[:, :, 2]], axis=1)
    pos8 = jnp.pad(pos, ((0, 0), (0, 5)))
    w1c = W1 * np.float32(1.0 / np.sqrt(NUM_RBF))
    w2c = W2 * np.float32(1.0 / np.sqrt(HIDDEN))
    w3c = W3 * np.float32(_ALPHA / np.sqrt(HIDDEN))

    posr = pos8[row]
    posc = pos8[col]
    xg = xT[row]
    msg = _edge_tc(posr, posc, xg, w1c, w2c, w3c)
    out = jax.ops.segment_sum(msg, row, num_segments=N_NODES)
    return _act_tc(out)


# speedup vs baseline: 4.3194x; 2.5511x over previous
import functools

import jax
import jax.numpy as jnp
import numpy as np
from jax import lax
from jax.experimental import pallas as pl
from jax.experimental.pallas import tpu as pltpu
from jax.experimental.pallas import tpu_sc as plsc

N_NODES = 10000
E_EDGES = 160000
MUL_IN = 32
MUL_OUT = 8
D_IN = 128
D_OUT = 32
NUM_RBF = 24
R_CUT = 10.0
HIDDEN = 64
BE = 2000

_SQRT3 = np.sqrt(3.0).astype(np.float32)
_ALPHA = np.float32(1.0 / np.sqrt(2 * MUL_IN))


def _edge_body(posr_ref, posc_ref, xg_ref, w1_ref, w2_ref, w3_ref, msg_ref):
    f32 = jnp.float32
    vec = posr_ref[...] - posc_ref[...]
    r2 = jnp.sum(vec * vec, axis=1, keepdims=True)
    r = jnp.sqrt(r2)
    n = vec * (1.0 / (r + 1e-9))
    y1_0 = _SQRT3 * n[:, 1:2]
    y1_1 = _SQRT3 * n[:, 2:3]
    y1_2 = _SQRT3 * n[:, 0:1]

    spacing = R_CUT / (NUM_RBF - 1)
    offs = lax.broadcasted_iota(jnp.int32, (1, NUM_RBF), 1).astype(f32) * spacing
    t = (r - offs) * (1.0 / spacing)
    phi = jnp.exp(-0.5 * t * t)

    h = phi @ w1_ref[...]
    h = h * jax.nn.sigmoid(h)
    h = h @ w2_ref[...]
    h = h * jax.nn.sigmoid(h)
    w = h @ w3_ref[...]

    xs = xg_ref[:, 0:32]
    xv0 = xg_ref[:, 32:64]
    xv1 = xg_ref[:, 64:96]
    xv2 = xg_ref[:, 96:128]
    s1 = (xv0 * y1_0 + xv1 * y1_1 + xv2 * y1_2) * (1.0 / _SQRT3)

    iu = lax.broadcasted_iota(jnp.int32, (MUL_IN, MUL_IN * MUL_OUT), 0)
    ij = lax.broadcasted_iota(jnp.int32, (MUL_IN, MUL_IN * MUL_OUT), 1)
    Q = (ij // MUL_OUT == iu).astype(f32)
    pj = lax.broadcasted_iota(jnp.int32, (MUL_IN * MUL_OUT, MUL_OUT), 0)
    pw = lax.broadcasted_iota(jnp.int32, (MUL_IN * MUL_OUT, MUL_OUT), 1)
    P8 = (pj % MUL_OUT == pw).astype(f32)

    s0r = xs @ Q
    s1r = s1 @ Q
    w0 = w[:, 0:256]
    w1p = w[:, 256:512]
    w2p = w[:, 512:768]
    w3p = w[:, 768:1024]

    out_s = (w0 * s0r + w3p * s1r) @ P8
    va = (w1p * s0r) @ P8
    vb0 = (w2p * (xv0 @ Q)) @ P8
    vb1 = (w2p * (xv1 @ Q)) @ P8
    vb2 = (w2p * (xv2 @ Q)) @ P8
    vk = jnp.concatenate(
        [va * y1_0 + vb0, va * y1_1 + vb1, va * y1_2 + vb2], axis=1
    )
    sk = lax.broadcasted_iota(jnp.int32, (24, 24), 0)
    sc = lax.broadcasted_iota(jnp.int32, (24, 24), 1)
    S = ((sc % 3) * 8 + sc // 3 == sk).astype(f32)
    msg_v = vk @ S
    msg_ref[...] = jnp.concatenate([out_s, msg_v], axis=1)


@jax.jit
def _edge_tc(posr, posc, xg, w1c, w2c, w3c):
    grid = (E_EDGES // BE,)
    return pl.pallas_call(
        _edge_body,
        grid=grid,
        in_specs=[
            pl.BlockSpec((BE, 8), lambda i: (i, 0)),
            pl.BlockSpec((BE, 8), lambda i: (i, 0)),
            pl.BlockSpec((BE, D_IN), lambda i: (i, 0)),
            pl.BlockSpec((NUM_RBF, HIDDEN), lambda i: (0, 0)),
            pl.BlockSpec((HIDDEN, HIDDEN), lambda i: (0, 0)),
            pl.BlockSpec((HIDDEN, 4 * MUL_IN * MUL_OUT), lambda i: (0, 0)),
        ],
        out_specs=pl.BlockSpec((BE, D_OUT), lambda i: (i, 0)),
        out_shape=jax.ShapeDtypeStruct((E_EDGES, D_OUT), jnp.float32),
        compiler_params=pltpu.CompilerParams(
            dimension_semantics=("arbitrary",),
        ),
    )(posr, posc, xg, w1c, w2c, w3c)


def _act_body(p0_ref, p1_ref, out_ref):
    o = p0_ref[...] + p1_ref[...]
    s = o[:, :MUL_OUT]
    out_ref[...] = jnp.concatenate(
        [s * jax.nn.sigmoid(s), o[:, MUL_OUT:]], axis=1
    )


@jax.jit
def _act_tc(p0, p1):
    return pl.pallas_call(
        _act_body,
        out_shape=jax.ShapeDtypeStruct((N_NODES, D_OUT), jnp.float32),
    )(p0, p1)


CH = 128
NCH = E_EDGES // CH
_SC_MESH = dict(core_axis_name="c", subcore_axis_name="s")
NW = 32


def _gather_body(xT_hbm, row_hbm, xg_hbm, idxr_v, xbuf, semx):
    c = lax.axis_index("c")
    s = lax.axis_index("s")
    wid = s * 2 + c
    n_my = (NCH - wid + NW - 1) // NW

    def step(i, carry):
        base = (wid + NW * i) * CH
        pltpu.sync_copy(row_hbm.at[pl.ds(base, CH)], idxr_v)
        pltpu.async_copy(xT_hbm.at[idxr_v], xbuf, semx).wait()
        pltpu.sync_copy(xbuf, xg_hbm.at[pl.ds(base, CH)])
        return carry

    lax.fori_loop(0, n_my, step, 0)


@jax.jit
def _sc_gather(xT, row):
    f = pl.kernel(
        _gather_body,
        out_type=jax.ShapeDtypeStruct((E_EDGES, D_IN), jnp.float32),
        mesh=plsc.VectorSubcoreMesh(**_SC_MESH),
        scratch_types=[
            pltpu.VMEM((CH,), jnp.int32),
            pltpu.VMEM((CH, D_IN), jnp.float32),
            pltpu.SemaphoreType.DMA,
        ],
    )
    return f(xT, row)


def _pos_body(pos8_hbm, row_hbm, col_hbm, posr_hbm, posc_hbm,
              idxr_v, idxc_v, prbuf, pcbuf, semr, semc):
    c = lax.axis_index("c")
    s = lax.axis_index("s")
    wid = s * 2 + c
    n_my = (NCH - wid + NW - 1) // NW

    def step(i, carry):
        base = (wid + NW * i) * CH
        pltpu.sync_copy(row_hbm.at[pl.ds(base, CH)], idxr_v)
        pltpu.sync_copy(col_hbm.at[pl.ds(base, CH)], idxc_v)
        cpr = pltpu.async_copy(pos8_hbm.at[idxr_v], prbuf, semr)
        cpc = pltpu.async_copy(pos8_hbm.at[idxc_v], pcbuf, semc)
        cpr.wait()
        cpc.wait()
        pltpu.sync_copy(prbuf, posr_hbm.at[pl.ds(base, CH)])
        pltpu.sync_copy(pcbuf, posc_hbm.at[pl.ds(base, CH)])
        return carry

    lax.fori_loop(0, n_my, step, 0)


@jax.jit
def _sc_pos(pos8, row, col):
    f = pl.kernel(
        _pos_body,
        out_type=[
            jax.ShapeDtypeStruct((E_EDGES, 8), jnp.float32),
            jax.ShapeDtypeStruct((E_EDGES, 8), jnp.float32),
        ],
        mesh=plsc.VectorSubcoreMesh(**_SC_MESH),
        scratch_types=[
            pltpu.VMEM((CH,), jnp.int32),
            pltpu.VMEM((CH,), jnp.int32),
            pltpu.VMEM((CH, 8), jnp.float32),
            pltpu.VMEM((CH, 8), jnp.float32),
            pltpu.SemaphoreType.DMA,
            pltpu.SemaphoreType.DMA,
        ],
        compiler_params=pltpu.CompilerParams(use_tc_tiling_on_sc=False),
    )
    return f(pos8, row, col)


ROWS_PER_TILE = 624


def _scatter_body(msg_hbm, row_hbm, zeros_hbm, parts_hbm,
                  idx_v, mbuf, acc_sh, sem):
    c = lax.axis_index("c")
    s = lax.axis_index("s")
    wid = s * 2 + c
    n_my = (NCH - wid + NW - 1) // NW

    @pl.when(s == 0)
    def _():
        pltpu.sync_copy(zeros_hbm, acc_sh)

    plsc.subcore_barrier()

    def step(i, carry):
        base = (wid + NW * i) * CH
        pltpu.sync_copy(row_hbm.at[pl.ds(base, CH)], idx_v)
        pltpu.sync_copy(msg_hbm.at[pl.ds(base, CH)], mbuf)
        pltpu.sync_copy(mbuf, acc_sh.at[idx_v], add=True)
        return carry

    lax.fori_loop(0, n_my, step, 0)
    plsc.subcore_barrier()
    dst = c * N_NODES + s * ROWS_PER_TILE
    pltpu.sync_copy(acc_sh.at[pl.ds(s * ROWS_PER_TILE, ROWS_PER_TILE)],
                    parts_hbm.at[pl.ds(dst, ROWS_PER_TILE)])
    tail = 16 * ROWS_PER_TILE

    @pl.when(s == 0)
    def _():
        pltpu.sync_copy(acc_sh.at[pl.ds(tail, N_NODES - tail)],
                        parts_hbm.at[pl.ds(c * N_NODES + tail, N_NODES - tail)])


@jax.jit
def _sc_scatter(msg, row, zeros):
    f = pl.kernel(
        _scatter_body,
        out_type=jax.ShapeDtypeStruct((2 * N_NODES, D_OUT), jnp.float32),
        mesh=plsc.VectorSubcoreMesh(**_SC_MESH),
        scratch_types=[
            pltpu.VMEM((CH,), jnp.int32),
            pltpu.VMEM((CH, D_OUT), jnp.float32),
            pltpu.VMEM_SHARED((N_NODES, D_OUT), jnp.float32),
            pltpu.SemaphoreType.DMA,
        ],
        compiler_params=pltpu.CompilerParams(use_tc_tiling_on_sc=False),
    )
    return f(msg, row, zeros)


def kernel(x, pos, edge_index, W1, W2, W3):
    row = edge_index[0]
    col = edge_index[1]
    xs = x[:, :MUL_IN]
    xv = x[:, MUL_IN:].reshape(N_NODES, MUL_IN, 3)
    xT = jnp.concatenate([xs, xv[:, :, 0], xv[:, :, 1], xv[:, :, 2]], axis=1)
    pos8 = jnp.pad(pos, ((0, 0), (0, 5)))
    w1c = W1 * np.float32(1.0 / np.sqrt(NUM_RBF))
    w2c = W2 * np.float32(1.0 / np.sqrt(HIDDEN))
    w3c = W3 * np.float32(_ALPHA / np.sqrt(HIDDEN))

    xg = _sc_gather(xT, row)
    posr, posc = _sc_pos(pos8, row, col)
    msg = _edge_tc(posr, posc, xg, w1c, w2c, w3c)
    parts = _sc_scatter(msg, row, jnp.zeros((N_NODES, D_OUT), jnp.float32))
    return _act_tc(parts[:N_NODES], parts[N_NODES:])
